# Initial kernel scaffold; baseline (speedup 1.0000x reference)
import jax
import jax.numpy as jnp
from jax.experimental import pallas as pl


def kernel(hidden_states, group_router_weight, expert_router_weights):
    raise NotImplementedError("write your pallas kernel here")


# speedup vs baseline: 10.3754x; 10.3754x over previous
import jax
import jax.numpy as jnp
from jax.experimental import pallas as pl
from jax.experimental.pallas import tpu as pltpu

B = 2048
HIDDEN = 2048
NUM_GROUPS = 4
NUM_EXPERTS = 64
EPG = 16
BT = 256
NBLK = B // BT

NEG_INF = float("-inf")


def _router_kernel(x_ref, gwt_ref, ewt_ref,
                   all_ref, topk_ref, wts_ref, lvar_ref, ent_ref,
                   psum_ref, esum_ref):
    step = pl.program_id(0)
    x = x_ref[...]

    glogits = jnp.dot(x, gwt_ref[...], preferred_element_type=jnp.float32)
    elogits = jnp.dot(x, ewt_ref[...], preferred_element_type=jnp.float32)

    iota_g = jax.lax.broadcasted_iota(jnp.int32, (BT, NUM_GROUPS), 1)
    gv1 = jnp.max(glogits, axis=1, keepdims=True)
    gi1 = jnp.min(jnp.where(glogits == gv1, iota_g, NUM_GROUPS),
                  axis=1, keepdims=True)
    gmasked = jnp.where(iota_g == gi1, NEG_INF, glogits)
    gv2 = jnp.max(gmasked, axis=1, keepdims=True)
    gi2 = jnp.min(jnp.where(gmasked == gv2, iota_g, NUM_GROUPS),
                  axis=1, keepdims=True)

    ge = jnp.exp(gv2 - gv1)
    denom = 1.0 + ge
    logw1 = jnp.log(1.0 / denom + 1e-8)
    logw2 = jnp.log(ge / denom + 1e-8)

    iota_e = jax.lax.broadcasted_iota(jnp.int32, (BT, NUM_EXPERTS), 1)
    gid = iota_e // EPG
    bias = jnp.where(gid == gi1, logw1,
                     jnp.where(gid == gi2, logw2, NEG_INF))
    all64 = elogits + bias
    all_ref[...] = all64

    ev1 = jnp.max(all64, axis=1, keepdims=True)
    ei1 = jnp.min(jnp.where(all64 == ev1, iota_e, NUM_EXPERTS),
                  axis=1, keepdims=True)
    emasked = jnp.where(iota_e == ei1, NEG_INF, all64)
    ev2 = jnp.max(emasked, axis=1, keepdims=True)
    ei2 = jnp.min(jnp.where(emasked == ev2, iota_e, NUM_EXPERTS),
                  axis=1, keepdims=True)
    topk_ref[...] = jnp.concatenate([ei1, ei2], axis=1)

    ee = jnp.exp(ev2 - ev1)
    wdenom = 1.0 + ee
    wts_ref[...] = jnp.concatenate([1.0 / wdenom, ee / wdenom], axis=1)

    pexp = jnp.exp(all64 - ev1)
    psum = jnp.sum(pexp, axis=1, keepdims=True)
    probs = pexp / psum
    plogp = probs * jnp.log(probs + 1e-8)
    ent_blk = -jnp.sum(plogp)
    probs_col = jnp.sum(probs, axis=0, keepdims=True)

    @pl.when(step == 0)
    def _init():
        psum_ref[...] = probs_col
        esum_ref[...] = jnp.full((1, 1), ent_blk, jnp.float32)

    @pl.when(step != 0)
    def _acc():
        psum_ref[...] += probs_col
        esum_ref[...] += ent_blk

    @pl.when(step == NBLK - 1)
    def _finalize():
        load = psum_ref[...] / jnp.float32(B)
        mu = jnp.mean(load)
        lvar_ref[...] = jnp.mean((load - mu) ** 2).reshape(1, 1)
        ent_ref[...] = esum_ref[...] / jnp.float32(B)


def kernel(hidden_states, group_router_weight, expert_router_weights):
    gwt = group_router_weight.T
    ewt = expert_router_weights.reshape(NUM_EXPERTS, HIDDEN).T

    grid = (NBLK,)
    out = pl.pallas_call(
        _router_kernel,
        grid=grid,
        in_specs=[
            pl.BlockSpec((BT, HIDDEN), lambda i: (i, 0)),
            pl.BlockSpec((HIDDEN, NUM_GROUPS), lambda i: (0, 0)),
            pl.BlockSpec((HIDDEN, NUM_EXPERTS), lambda i: (0, 0)),
        ],
        out_specs=[
            pl.BlockSpec((BT, NUM_EXPERTS), lambda i: (i, 0)),
            pl.BlockSpec((BT, 2), lambda i: (i, 0)),
            pl.BlockSpec((BT, 2), lambda i: (i, 0)),
            pl.BlockSpec((1, 1), lambda i: (0, 0)),
            pl.BlockSpec((1, 1), lambda i: (0, 0)),
        ],
        out_shape=[
            jax.ShapeDtypeStruct((B, NUM_EXPERTS), jnp.float32),
            jax.ShapeDtypeStruct((B, 2), jnp.int32),
            jax.ShapeDtypeStruct((B, 2), jnp.float32),
            jax.ShapeDtypeStruct((1, 1), jnp.float32),
            jax.ShapeDtypeStruct((1, 1), jnp.float32),
        ],
        scratch_shapes=[
            pltpu.VMEM((1, NUM_EXPERTS), jnp.float32),
            pltpu.VMEM((1, 1), jnp.float32),
        ],
    )(hidden_states, gwt, ewt)
    all64, topk, wts, lvar, ent = out
    return (all64, topk, wts, lvar.reshape(()), ent.reshape(()))


# speedup vs baseline: 10.9430x; 1.0547x over previous
import jax
import jax.numpy as jnp
from jax.experimental import pallas as pl
from jax.experimental.pallas import tpu as pltpu

B = 2048
HIDDEN = 2048
NUM_GROUPS = 4
NUM_EXPERTS = 64
EPG = 16
BT = 256
NBLK = B // BT

NEG_INF = float("-inf")


def _router_kernel(x_ref, wt_ref,
                   all_ref, topk_ref, wts_ref, lvar_ref, ent_ref,
                   psum_ref, esum_ref):
    step = pl.program_id(0)
    x = x_ref[...]

    logits = jnp.dot(x, wt_ref[...], preferred_element_type=jnp.float32)
    elogits = logits[:, :NUM_EXPERTS]

    g0 = logits[:, NUM_EXPERTS + 0:NUM_EXPERTS + 1]
    g1 = logits[:, NUM_EXPERTS + 1:NUM_EXPERTS + 2]
    g2 = logits[:, NUM_EXPERTS + 2:NUM_EXPERTS + 3]
    g3 = logits[:, NUM_EXPERTS + 3:NUM_EXPERTS + 4]
    p01 = g0 >= g1
    m01v = jnp.where(p01, g0, g1)
    m01i = jnp.where(p01, 0, 1)
    s01v = jnp.where(p01, g1, g0)
    s01i = jnp.where(p01, 1, 0)
    p23 = g2 >= g3
    m23v = jnp.where(p23, g2, g3)
    m23i = jnp.where(p23, 2, 3)
    s23v = jnp.where(p23, g3, g2)
    s23i = jnp.where(p23, 3, 2)
    pw = m01v >= m23v
    gv1 = jnp.where(pw, m01v, m23v)
    gi1 = jnp.where(pw, m01i, m23i)
    c2v = jnp.where(pw, s01v, m01v)
    c2i = jnp.where(pw, s01i, m01i)
    c3v = jnp.where(pw, m23v, s23v)
    c3i = jnp.where(pw, m23i, s23i)
    p2 = c2v >= c3v
    gv2 = jnp.where(p2, c2v, c3v)
    gi2 = jnp.where(p2, c2i, c3i)

    ge = jnp.exp(gv2 - gv1)
    denom = 1.0 + ge
    logw1 = jnp.log(1.0 / denom + 1e-8)
    logw2 = jnp.log(ge / denom + 1e-8)

    iota_e = jax.lax.broadcasted_iota(jnp.int32, (BT, NUM_EXPERTS), 1)
    gid = iota_e // EPG
    bias = jnp.where(gid == gi1, logw1,
                     jnp.where(gid == gi2, logw2, NEG_INF))
    all64 = elogits + bias
    all_ref[...] = all64

    iota_col = jax.lax.broadcasted_iota(
        jnp.int32, (NUM_EXPERTS, 1), 0).astype(jnp.float32)
    ev1 = jnp.max(all64, axis=1, keepdims=True)
    eq1 = (all64 == ev1).astype(jnp.float32)
    ei1f = jnp.dot(eq1, iota_col, preferred_element_type=jnp.float32)
    ei1 = ei1f.astype(jnp.int32)
    emasked = jnp.where(iota_e == ei1, NEG_INF, all64)
    ev2 = jnp.max(emasked, axis=1, keepdims=True)
    eq2 = (emasked == ev2).astype(jnp.float32)
    ei2f = jnp.dot(eq2, iota_col, preferred_element_type=jnp.float32)
    ei2 = ei2f.astype(jnp.int32)
    topk_ref[...] = jnp.concatenate([ei1, ei2], axis=1)

    ee = jnp.exp(ev2 - ev1)
    wdenom = 1.0 + ee
    wts_ref[...] = jnp.concatenate([1.0 / wdenom, ee / wdenom], axis=1)

    ones_col = jnp.ones((NUM_EXPERTS, 1), jnp.float32)
    ones_row = jnp.ones((1, BT), jnp.float32)
    pexp = jnp.exp(all64 - ev1)
    psum = jnp.dot(pexp, ones_col, preferred_element_type=jnp.float32)
    probs = pexp / psum
    plogp = probs * jnp.log(probs + 1e-8)
    probs_col = jnp.dot(ones_row, probs, preferred_element_type=jnp.float32)
    ent_col = jnp.dot(ones_row, plogp, preferred_element_type=jnp.float32)

    @pl.when(step == 0)
    def _init():
        psum_ref[...] = probs_col
        esum_ref[...] = ent_col

    @pl.when(step != 0)
    def _acc():
        psum_ref[...] += probs_col
        esum_ref[...] += ent_col

    @pl.when(step == NBLK - 1)
    def _finalize():
        load = psum_ref[...] / jnp.float32(B)
        mu = jnp.mean(load)
        lvar_ref[...] = jnp.mean((load - mu) ** 2).reshape(1, 1)
        ent_ref[...] = (-jnp.sum(esum_ref[...]) / jnp.float32(B)).reshape(1, 1)


def kernel(hidden_states, group_router_weight, expert_router_weights):
    ewt = expert_router_weights.reshape(NUM_EXPERTS, HIDDEN)
    wt = jnp.concatenate([ewt, group_router_weight], axis=0).T

    grid = (NBLK,)
    out = pl.pallas_call(
        _router_kernel,
        grid=grid,
        in_specs=[
            pl.BlockSpec((BT, HIDDEN), lambda i: (i, 0)),
            pl.BlockSpec((HIDDEN, NUM_EXPERTS + NUM_GROUPS), lambda i: (0, 0)),
        ],
        out_specs=[
            pl.BlockSpec((BT, NUM_EXPERTS), lambda i: (i, 0)),
            pl.BlockSpec((BT, 2), lambda i: (i, 0)),
            pl.BlockSpec((BT, 2), lambda i: (i, 0)),
            pl.BlockSpec((1, 1), lambda i: (0, 0)),
            pl.BlockSpec((1, 1), lambda i: (0, 0)),
        ],
        out_shape=[
            jax.ShapeDtypeStruct((B, NUM_EXPERTS), jnp.float32),
            jax.ShapeDtypeStruct((B, 2), jnp.int32),
            jax.ShapeDtypeStruct((B, 2), jnp.float32),
            jax.ShapeDtypeStruct((1, 1), jnp.float32),
            jax.ShapeDtypeStruct((1, 1), jnp.float32),
        ],
        scratch_shapes=[
            pltpu.VMEM((1, NUM_EXPERTS), jnp.float32),
            pltpu.VMEM((1, NUM_EXPERTS), jnp.float32),
        ],
    )(hidden_states, wt)
    all64, topk, wts, lvar, ent = out
    return (all64, topk, wts, lvar.reshape(()), ent.reshape(()))


# speedup vs baseline: 11.8970x; 1.0872x over previous
import jax
import jax.numpy as jnp
from jax.experimental import pallas as pl
from jax.experimental.pallas import tpu as pltpu

B = 2048
HIDDEN = 2048
NUM_GROUPS = 4
NUM_EXPERTS = 64
EPG = 16
BT = 256
NBLK = B // BT

NEG_INF = float("-inf")


def _router_kernel(x_ref, wt_ref,
                   all_ref, topk_ref, wts_ref, lvar_ref, ent_ref,
                   psum_ref, esum_ref):
    step = pl.program_id(0)
    x = x_ref[...]

    logits = jax.lax.dot_general(
        x, wt_ref[...], (((1,), (1,)), ((), ())),
        preferred_element_type=jnp.float32)
    elogits = logits[:, :NUM_EXPERTS]

    g0 = logits[:, NUM_EXPERTS + 0:NUM_EXPERTS + 1]
    g1 = logits[:, NUM_EXPERTS + 1:NUM_EXPERTS + 2]
    g2 = logits[:, NUM_EXPERTS + 2:NUM_EXPERTS + 3]
    g3 = logits[:, NUM_EXPERTS + 3:NUM_EXPERTS + 4]
    p01 = g0 >= g1
    m01v = jnp.where(p01, g0, g1)
    m01i = jnp.where(p01, 0, 1)
    s01v = jnp.where(p01, g1, g0)
    s01i = jnp.where(p01, 1, 0)
    p23 = g2 >= g3
    m23v = jnp.where(p23, g2, g3)
    m23i = jnp.where(p23, 2, 3)
    s23v = jnp.where(p23, g3, g2)
    s23i = jnp.where(p23, 3, 2)
    pw = m01v >= m23v
    gv1 = jnp.where(pw, m01v, m23v)
    gi1 = jnp.where(pw, m01i, m23i)
    c2v = jnp.where(pw, s01v, m01v)
    c2i = jnp.where(pw, s01i, m01i)
    c3v = jnp.where(pw, m23v, s23v)
    c3i = jnp.where(pw, m23i, s23i)
    p2 = c2v >= c3v
    gv2 = jnp.where(p2, c2v, c3v)
    gi2 = jnp.where(p2, c2i, c3i)

    ge = jnp.exp(gv2 - gv1)
    denom = 1.0 + ge
    logw1 = jnp.log(1.0 / denom + 1e-8)
    logw2 = jnp.log(ge / denom + 1e-8)

    iota_e = jax.lax.broadcasted_iota(jnp.int32, (BT, NUM_EXPERTS), 1)
    gid = iota_e // EPG
    bias = jnp.where(gid == gi1, logw1,
                     jnp.where(gid == gi2, logw2, NEG_INF))
    all64 = elogits + bias
    all_ref[...] = all64

    iota_col = jax.lax.broadcasted_iota(
        jnp.int32, (NUM_EXPERTS, 1), 0).astype(jnp.float32)
    ev1 = jnp.max(all64, axis=1, keepdims=True)
    eq1 = (all64 == ev1).astype(jnp.float32)
    ei1f = jnp.dot(eq1, iota_col, preferred_element_type=jnp.float32)
    ei1 = ei1f.astype(jnp.int32)
    emasked = jnp.where(iota_e == ei1, NEG_INF, all64)
    ev2 = jnp.max(emasked, axis=1, keepdims=True)
    eq2 = (emasked == ev2).astype(jnp.float32)
    ei2f = jnp.dot(eq2, iota_col, preferred_element_type=jnp.float32)
    ei2 = ei2f.astype(jnp.int32)
    topk_ref[...] = jnp.concatenate([ei1, ei2], axis=1)

    ee = jnp.exp(ev2 - ev1)
    wdenom = 1.0 + ee
    wts_ref[...] = jnp.concatenate([1.0 / wdenom, ee / wdenom], axis=1)

    ones_col = jnp.ones((NUM_EXPERTS, 1), jnp.float32)
    ones_row = jnp.ones((1, BT), jnp.float32)
    pexp = jnp.exp(all64 - ev1)
    psum = jnp.dot(pexp, ones_col, preferred_element_type=jnp.float32)
    probs = pexp / psum
    plogp = probs * jnp.log(probs + 1e-8)
    probs_col = jnp.dot(ones_row, probs, preferred_element_type=jnp.float32)
    ent_col = jnp.dot(ones_row, plogp, preferred_element_type=jnp.float32)

    @pl.when(step == 0)
    def _init():
        psum_ref[...] = probs_col
        esum_ref[...] = ent_col

    @pl.when(step != 0)
    def _acc():
        psum_ref[...] += probs_col
        esum_ref[...] += ent_col

    @pl.when(step == NBLK - 1)
    def _finalize():
        load = psum_ref[...] / jnp.float32(B)
        mu = jnp.mean(load)
        lvar_ref[...] = jnp.mean((load - mu) ** 2).reshape(1, 1)
        ent_ref[...] = (-jnp.sum(esum_ref[...]) / jnp.float32(B)).reshape(1, 1)


def kernel(hidden_states, group_router_weight, expert_router_weights):
    ewt = expert_router_weights.reshape(NUM_EXPERTS, HIDDEN)
    wt = jnp.concatenate([ewt, group_router_weight], axis=0)

    grid = (NBLK,)
    out = pl.pallas_call(
        _router_kernel,
        grid=grid,
        in_specs=[
            pl.BlockSpec((BT, HIDDEN), lambda i: (i, 0)),
            pl.BlockSpec((NUM_EXPERTS + NUM_GROUPS, HIDDEN), lambda i: (0, 0)),
        ],
        out_specs=[
            pl.BlockSpec((BT, NUM_EXPERTS), lambda i: (i, 0)),
            pl.BlockSpec((BT, 2), lambda i: (i, 0)),
            pl.BlockSpec((BT, 2), lambda i: (i, 0)),
            pl.BlockSpec((1, 1), lambda i: (0, 0)),
            pl.BlockSpec((1, 1), lambda i: (0, 0)),
        ],
        out_shape=[
            jax.ShapeDtypeStruct((B, NUM_EXPERTS), jnp.float32),
            jax.ShapeDtypeStruct((B, 2), jnp.int32),
            jax.ShapeDtypeStruct((B, 2), jnp.float32),
            jax.ShapeDtypeStruct((1, 1), jnp.float32),
            jax.ShapeDtypeStruct((1, 1), jnp.float32),
        ],
        scratch_shapes=[
            pltpu.VMEM((1, NUM_EXPERTS), jnp.float32),
            pltpu.VMEM((1, NUM_EXPERTS), jnp.float32),
        ],
    )(hidden_states, wt)
    all64, topk, wts, lvar, ent = out
    return (all64, topk, wts, lvar.reshape(()), ent.reshape(()))


# speedup vs baseline: 12.8606x; 1.0810x over previous
import jax
import jax.numpy as jnp
from jax.experimental import pallas as pl
from jax.experimental.pallas import tpu as pltpu

B = 2048
HIDDEN = 2048
NUM_GROUPS = 4
NUM_EXPERTS = 64
EPG = 16
BT = 512
NBLK = B // BT

NEG_INF = float("-inf")


def _router_kernel(x_ref, wt_ref,
                   all_ref, topk_ref, wts_ref, lvar_ref, ent_ref,
                   psum_ref, esum_ref):
    step = pl.program_id(0)
    x = x_ref[...]

    logits = jax.lax.dot_general(
        x, wt_ref[...], (((1,), (1,)), ((), ())),
        preferred_element_type=jnp.float32)
    elogits = logits[:, :NUM_EXPERTS]

    g0 = logits[:, NUM_EXPERTS + 0:NUM_EXPERTS + 1]
    g1 = logits[:, NUM_EXPERTS + 1:NUM_EXPERTS + 2]
    g2 = logits[:, NUM_EXPERTS + 2:NUM_EXPERTS + 3]
    g3 = logits[:, NUM_EXPERTS + 3:NUM_EXPERTS + 4]
    p01 = g0 >= g1
    m01v = jnp.where(p01, g0, g1)
    m01i = jnp.where(p01, 0, 1)
    s01v = jnp.where(p01, g1, g0)
    s01i = jnp.where(p01, 1, 0)
    p23 = g2 >= g3
    m23v = jnp.where(p23, g2, g3)
    m23i = jnp.where(p23, 2, 3)
    s23v = jnp.where(p23, g3, g2)
    s23i = jnp.where(p23, 3, 2)
    pw = m01v >= m23v
    gv1 = jnp.where(pw, m01v, m23v)
    gi1 = jnp.where(pw, m01i, m23i)
    c2v = jnp.where(pw, s01v, m01v)
    c2i = jnp.where(pw, s01i, m01i)
    c3v = jnp.where(pw, m23v, s23v)
    c3i = jnp.where(pw, m23i, s23i)
    p2 = c2v >= c3v
    gv2 = jnp.where(p2, c2v, c3v)
    gi2 = jnp.where(p2, c2i, c3i)

    ge = jnp.exp(gv2 - gv1)
    denom = 1.0 + ge
    logw1 = jnp.log(1.0 / denom + 1e-8)
    logw2 = jnp.log(ge / denom + 1e-8)

    iota_e = jax.lax.broadcasted_iota(jnp.int32, (BT, NUM_EXPERTS), 1)
    gid = iota_e // EPG
    bias = jnp.where(gid == gi1, logw1,
                     jnp.where(gid == gi2, logw2, NEG_INF))
    all64 = elogits + bias
    all_ref[...] = all64

    iota_col = jax.lax.broadcasted_iota(
        jnp.int32, (NUM_EXPERTS, 1), 0).astype(jnp.float32)
    ev1 = jnp.max(all64, axis=1, keepdims=True)
    eq1 = (all64 == ev1).astype(jnp.float32)
    ei1f = jnp.dot(eq1, iota_col, preferred_element_type=jnp.float32)
    ei1 = ei1f.astype(jnp.int32)
    emasked = jnp.where(iota_e == ei1, NEG_INF, all64)
    ev2 = jnp.max(emasked, axis=1, keepdims=True)
    eq2 = (emasked == ev2).astype(jnp.float32)
    ei2f = jnp.dot(eq2, iota_col, preferred_element_type=jnp.float32)
    ei2 = ei2f.astype(jnp.int32)
    topk_ref[...] = jnp.concatenate([ei1, ei2], axis=1)

    ee = jnp.exp(ev2 - ev1)
    wdenom = 1.0 + ee
    wts_ref[...] = jnp.concatenate([1.0 / wdenom, ee / wdenom], axis=1)

    ones_col = jnp.ones((NUM_EXPERTS, 1), jnp.float32)
    ones_row = jnp.ones((1, BT), jnp.float32)
    pexp = jnp.exp(all64 - ev1)
    psum = jnp.dot(pexp, ones_col, preferred_element_type=jnp.float32)
    probs = pexp / psum
    plogp = probs * jnp.log(probs + 1e-8)
    probs_col = jnp.dot(ones_row, probs, preferred_element_type=jnp.float32)
    ent_col = jnp.dot(ones_row, plogp, preferred_element_type=jnp.float32)

    @pl.when(step == 0)
    def _init():
        psum_ref[...] = probs_col
        esum_ref[...] = ent_col

    @pl.when(step != 0)
    def _acc():
        psum_ref[...] += probs_col
        esum_ref[...] += ent_col

    @pl.when(step == NBLK - 1)
    def _finalize():
        load = psum_ref[...] / jnp.float32(B)
        mu = jnp.mean(load)
        lvar_ref[...] = jnp.mean((load - mu) ** 2).reshape(1, 1)
        ent_ref[...] = (-jnp.sum(esum_ref[...]) / jnp.float32(B)).reshape(1, 1)


def kernel(hidden_states, group_router_weight, expert_router_weights):
    ewt = expert_router_weights.reshape(NUM_EXPERTS, HIDDEN)
    wt = jnp.concatenate([ewt, group_router_weight], axis=0)

    grid = (NBLK,)
    out = pl.pallas_call(
        _router_kernel,
        grid=grid,
        in_specs=[
            pl.BlockSpec((BT, HIDDEN), lambda i: (i, 0)),
            pl.BlockSpec((NUM_EXPERTS + NUM_GROUPS, HIDDEN), lambda i: (0, 0)),
        ],
        out_specs=[
            pl.BlockSpec((BT, NUM_EXPERTS), lambda i: (i, 0)),
            pl.BlockSpec((BT, 2), lambda i: (i, 0)),
            pl.BlockSpec((BT, 2), lambda i: (i, 0)),
            pl.BlockSpec((1, 1), lambda i: (0, 0)),
            pl.BlockSpec((1, 1), lambda i: (0, 0)),
        ],
        out_shape=[
            jax.ShapeDtypeStruct((B, NUM_EXPERTS), jnp.float32),
            jax.ShapeDtypeStruct((B, 2), jnp.int32),
            jax.ShapeDtypeStruct((B, 2), jnp.float32),
            jax.ShapeDtypeStruct((1, 1), jnp.float32),
            jax.ShapeDtypeStruct((1, 1), jnp.float32),
        ],
        scratch_shapes=[
            pltpu.VMEM((1, NUM_EXPERTS), jnp.float32),
            pltpu.VMEM((1, NUM_EXPERTS), jnp.float32),
        ],
    )(hidden_states, wt)
    all64, topk, wts, lvar, ent = out
    return (all64, topk, wts, lvar.reshape(()), ent.reshape(()))
